# Initial kernel scaffold; baseline (speedup 1.0000x reference)
import functools

import jax
import jax.numpy as jnp
from jax import lax
from jax.experimental import pallas as pl
from jax.experimental.pallas import tpu as pltpu
from jax.experimental.pallas import tpu_sc as plsc

N_ATOMS = 2097152
N_TYPES = 100
N_SYSTEMS = 8192

NUM_CORES = 2
NUM_SUBCORES = 16
NW = NUM_CORES * NUM_SUBCORES
ATOMS_PER_W = N_ATOMS // NW
CHUNK = 16384
NCHUNKS = ATOMS_PER_W // CHUNK
VECS = CHUNK // 16
PAD = 8
WPAD = 128


def _sc_body(types_hbm, sys_hbm, w_hbm, part_hbm, wbuf, tbuf, sbuf, acc):
  wid = lax.axis_index("s") * NUM_CORES + lax.axis_index("c")
  base = wid * ATOMS_PER_W

  pltpu.sync_copy(w_hbm, wbuf)

  def zero_body(i, c):
    acc[pl.ds(i * 16, 16)] = jnp.zeros((16,), jnp.float32)
    return c

  lax.fori_loop(0, N_SYSTEMS // 16, zero_body, 0)

  iota = lax.iota(jnp.int32, 16)
  lane_first = iota == 0
  lane_last = iota == 15

  def chunk_body(cidx, carry):
    cbase = base + cidx * CHUNK
    pltpu.sync_copy(types_hbm.at[pl.ds(cbase, CHUNK)], tbuf)
    pltpu.sync_copy(sys_hbm.at[pl.ds(cbase, CHUNK)], sbuf.at[pl.ds(PAD, CHUNK)])
    look = jnp.minimum(cbase + CHUNK, N_ATOMS - 16)
    pltpu.sync_copy(sys_hbm.at[pl.ds(look, 16)], sbuf.at[pl.ds(PAD + CHUNK, 16)])

    def vec_body(k, c):
      off = k * 16 + PAD
      s = sbuf[pl.ds(off, 16)]
      s_prev = sbuf[pl.ds(off - 1, 16)]
      s_next = sbuf[pl.ds(off + 1, 16)]
      t = tbuf[pl.ds(k * 16, 16)]
      vw = plsc.load_gather(wbuf, [t])
      cs = plsc.cumsum(vw)
      run_end = (s != s_next) | lane_last
      run_start = (s != s_prev) | lane_first
      plsc.addupdate_scatter(acc, [s], cs, mask=run_end)
      plsc.addupdate_scatter(acc, [s], vw - cs, mask=run_start)
      return c

    lax.fori_loop(0, VECS, vec_body, carry)
    return carry

  lax.fori_loop(0, NCHUNKS, chunk_body, 0)
  pltpu.sync_copy(acc, part_hbm.at[wid])


def _merge_body(p_ref, o_ref):
  o_ref[...] = jnp.sum(p_ref[...], axis=0, keepdims=True)


@jax.jit
def kernel(types, system_ids, weights):
  w_pad = jnp.zeros((WPAD,), jnp.float32).at[:N_TYPES].set(weights)

  sc_fn = pl.kernel(
      _sc_body,
      out_type=jax.ShapeDtypeStruct((NW, N_SYSTEMS), jnp.float32),
      mesh=plsc.VectorSubcoreMesh(core_axis_name="c", subcore_axis_name="s"),
      scratch_types=[
          pltpu.VMEM((WPAD,), jnp.float32),
          pltpu.VMEM((CHUNK,), jnp.int32),
          pltpu.VMEM((PAD + CHUNK + 16,), jnp.int32),
          pltpu.VMEM((N_SYSTEMS,), jnp.float32),
      ],
  )
  partials = sc_fn(types, system_ids, w_pad)

  merged = pl.pallas_call(
      _merge_body,
      out_shape=jax.ShapeDtypeStruct((1, N_SYSTEMS), jnp.float32),
  )(partials)
  return merged.reshape(N_SYSTEMS, 1)


# speedup vs baseline: 497.5992x; 497.5992x over previous
import jax
import jax.numpy as jnp
from jax import lax
from jax.experimental import pallas as pl
from jax.experimental.pallas import tpu as pltpu
from jax.experimental.pallas import tpu_sc as plsc

N_ATOMS = 2097152
N_TYPES = 100
N_SYSTEMS = 8192

NUM_CORES = 2
NUM_SUBCORES = 16
NW = NUM_CORES * NUM_SUBCORES
ATOMS_PER_W = N_ATOMS // NW
CHUNK = 16384
NCHUNKS = ATOMS_PER_W // CHUNK
VECS = CHUNK // 16
PAD = 8
WPAD = 128
UNROLL = 8


def _sc_body(types_hbm, sys_hbm, w_hbm, part_hbm, wbuf, tbuf0, tbuf1,
             sbuf0, sbuf1, acc, sem_t0, sem_t1, sem_s0, sem_s1):
  tbufs = (tbuf0, tbuf1)
  sbufs = (sbuf0, sbuf1)
  sem_ts = (sem_t0, sem_t1)
  sem_ss = (sem_s0, sem_s1)
  wid = lax.axis_index("s") * NUM_CORES + lax.axis_index("c")
  base = wid * ATOMS_PER_W

  pltpu.sync_copy(w_hbm, wbuf)

  @plsc.parallel_loop(0, N_SYSTEMS // 16, 1, unroll=8)
  def zero_body(i):
    acc[pl.ds(i * 16, 16)] = jnp.zeros((16,), jnp.float32)

  iota = lax.iota(jnp.int32, 16)
  lane_first = iota == 0
  lane_last = iota == 15

  def start_chunk(c):
    slot = c % 2
    cbase = base + c * CHUNK
    d1 = pltpu.async_copy(
        types_hbm.at[pl.ds(cbase, CHUNK)], tbufs[slot], sem_ts[slot])
    d2 = pltpu.async_copy(
        sys_hbm.at[pl.ds(cbase, CHUNK)],
        sbufs[slot].at[pl.ds(PAD, CHUNK)], sem_ss[slot])
    look = jnp.minimum(cbase + CHUNK, N_ATOMS - 16)
    d3 = pltpu.async_copy(
        sys_hbm.at[pl.ds(look, 16)],
        sbufs[slot].at[pl.ds(PAD + CHUNK, 16)], sem_ss[slot])
    return (d1, d2, d3)

  def compute(slot):
    tb = tbufs[slot]
    sb = sbufs[slot]

    @plsc.parallel_loop(0, VECS, 1, unroll=UNROLL)
    def vec_body(k):
      off = k * 16 + PAD
      s = sb[pl.ds(off, 16)]
      s_prev = sb[pl.ds(off - 1, 16)]
      s_next = sb[pl.ds(off + 1, 16)]
      t = tb[pl.ds(k * 16, 16)]
      vw = plsc.load_gather(wbuf, [t])
      cs = plsc.cumsum(vw)
      run_end = (s != s_next) | lane_last
      run_start = (s != s_prev) | lane_first
      plsc.addupdate_scatter(acc, [s], cs, mask=run_end)
      plsc.addupdate_scatter(acc, [s], vw - cs, mask=run_start)

  pending = start_chunk(0)
  for c in range(NCHUNKS):
    nxt = start_chunk(c + 1) if c + 1 < NCHUNKS else None
    for d in pending:
      d.wait()
    compute(c % 2)
    pending = nxt

  pltpu.sync_copy(acc, part_hbm.at[wid])


def _merge_body(p_ref, o_ref):
  o_ref[...] = jnp.sum(p_ref[...], axis=0, keepdims=True)


@jax.jit
def kernel(types, system_ids, weights):
  w_pad = jnp.zeros((WPAD,), jnp.float32).at[:N_TYPES].set(weights)

  sc_fn = pl.kernel(
      _sc_body,
      out_type=jax.ShapeDtypeStruct((NW, N_SYSTEMS), jnp.float32),
      mesh=plsc.VectorSubcoreMesh(core_axis_name="c", subcore_axis_name="s"),
      compiler_params=pltpu.CompilerParams(needs_layout_passes=False),
      scratch_types=[
          pltpu.VMEM((WPAD,), jnp.float32),
          pltpu.VMEM((CHUNK,), jnp.int32),
          pltpu.VMEM((CHUNK,), jnp.int32),
          pltpu.VMEM((PAD + CHUNK + 16,), jnp.int32),
          pltpu.VMEM((PAD + CHUNK + 16,), jnp.int32),
          pltpu.VMEM((N_SYSTEMS,), jnp.float32),
          pltpu.SemaphoreType.DMA,
          pltpu.SemaphoreType.DMA,
          pltpu.SemaphoreType.DMA,
          pltpu.SemaphoreType.DMA,
      ],
  )
  partials = sc_fn(types, system_ids, w_pad)

  merged = pl.pallas_call(
      _merge_body,
      out_shape=jax.ShapeDtypeStruct((1, N_SYSTEMS), jnp.float32),
  )(partials)
  return merged.reshape(N_SYSTEMS, 1)


# speedup vs baseline: 557.2268x; 1.1198x over previous
import jax
import jax.numpy as jnp
from jax import lax
from jax.experimental import pallas as pl
from jax.experimental.pallas import tpu as pltpu
from jax.experimental.pallas import tpu_sc as plsc

N_ATOMS = 2097152
N_TYPES = 100
N_SYSTEMS = 8192

NUM_CORES = 2
NUM_SUBCORES = 16
NW = NUM_CORES * NUM_SUBCORES
ATOMS_PER_W = N_ATOMS // NW
CHUNK = 16384
NCHUNKS = ATOMS_PER_W // CHUNK
VECS = CHUNK // 16
WPAD = 128
UNROLL = 8


def _sc_body(types_hbm, sys_hbm, w_hbm, part_hbm, wbuf, tbuf0, tbuf1,
             sbuf0, sbuf1, acc, sem_t0, sem_t1, sem_s0, sem_s1):
  tbufs = (tbuf0, tbuf1)
  sbufs = (sbuf0, sbuf1)
  sem_ts = (sem_t0, sem_t1)
  sem_ss = (sem_s0, sem_s1)
  wid = lax.axis_index("s") * NUM_CORES + lax.axis_index("c")
  base = wid * ATOMS_PER_W

  pltpu.sync_copy(w_hbm, wbuf)

  @plsc.parallel_loop(0, N_SYSTEMS // 16, 1, unroll=8)
  def zero_body(i):
    acc[pl.ds(i * 16, 16)] = jnp.zeros((16,), jnp.float32)

  iota = lax.iota(jnp.int32, 16)
  lane_first = iota == 0
  lane_last = iota == 15
  key_rotl = (iota + 15) & 15
  key_rotr = (iota + 1) & 15

  def start_chunk(c):
    slot = c % 2
    cbase = base + c * CHUNK
    d1 = pltpu.async_copy(
        types_hbm.at[pl.ds(cbase, CHUNK)], tbufs[slot], sem_ts[slot])
    d2 = pltpu.async_copy(
        sys_hbm.at[pl.ds(cbase, CHUNK)], sbufs[slot], sem_ss[slot])
    return (d1, d2)

  def compute(slot):
    tb = tbufs[slot]
    sb = sbufs[slot]

    @plsc.parallel_loop(0, VECS, 1, unroll=UNROLL)
    def vec_body(k):
      s = sb[pl.ds(k * 16, 16)]
      _, s_next = plsc.sort_key_val(key_rotl, s)
      _, s_prev = plsc.sort_key_val(key_rotr, s)
      t = tb[pl.ds(k * 16, 16)]
      vw = plsc.load_gather(wbuf, [t])
      cs = plsc.cumsum(vw)
      run_end = (s != s_next) | lane_last
      run_start = (s != s_prev) | lane_first
      plsc.addupdate_scatter(acc, [s], cs, mask=run_end)
      plsc.addupdate_scatter(acc, [s], vw - cs, mask=run_start)

  pending = start_chunk(0)
  for c in range(NCHUNKS):
    nxt = start_chunk(c + 1) if c + 1 < NCHUNKS else None
    for d in pending:
      d.wait()
    compute(c % 2)
    pending = nxt

  pltpu.sync_copy(acc, part_hbm.at[wid])


def _merge_body(p_ref, o_ref):
  o_ref[...] = jnp.sum(p_ref[...], axis=0, keepdims=True)


@jax.jit
def kernel(types, system_ids, weights):
  w_pad = jnp.zeros((WPAD,), jnp.float32).at[:N_TYPES].set(weights)

  sc_fn = pl.kernel(
      _sc_body,
      out_type=jax.ShapeDtypeStruct((NW, N_SYSTEMS), jnp.float32),
      mesh=plsc.VectorSubcoreMesh(core_axis_name="c", subcore_axis_name="s"),
      compiler_params=pltpu.CompilerParams(needs_layout_passes=False),
      scratch_types=[
          pltpu.VMEM((WPAD,), jnp.float32),
          pltpu.VMEM((CHUNK,), jnp.int32),
          pltpu.VMEM((CHUNK,), jnp.int32),
          pltpu.VMEM((CHUNK,), jnp.int32),
          pltpu.VMEM((CHUNK,), jnp.int32),
          pltpu.VMEM((N_SYSTEMS,), jnp.float32),
          pltpu.SemaphoreType.DMA,
          pltpu.SemaphoreType.DMA,
          pltpu.SemaphoreType.DMA,
          pltpu.SemaphoreType.DMA,
      ],
  )
  partials = sc_fn(types, system_ids, w_pad)

  merged = pl.pallas_call(
      _merge_body,
      out_shape=jax.ShapeDtypeStruct((1, N_SYSTEMS), jnp.float32),
  )(partials)
  return merged.reshape(N_SYSTEMS, 1)


# speedup vs baseline: 571.2570x; 1.0252x over previous
import jax
import jax.numpy as jnp
from jax import lax
from jax.experimental import pallas as pl
from jax.experimental.pallas import tpu as pltpu
from jax.experimental.pallas import tpu_sc as plsc

N_ATOMS = 2097152
N_TYPES = 100
N_SYSTEMS = 8192

NUM_CORES = 2
NUM_SUBCORES = 16
NW = NUM_CORES * NUM_SUBCORES
ATOMS_PER_W = N_ATOMS // NW
CHUNK = 16384
NCHUNKS = ATOMS_PER_W // CHUNK
VECS = CHUNK // 16
WPAD = 128
UNROLL = 8


def _sc_body(types_hbm, sys_hbm, w_hbm, part_hbm, wbuf, tbuf0, tbuf1,
             sbuf0, sbuf1, acc, sem_t0, sem_t1, sem_s0, sem_s1):
  tbufs = (tbuf0, tbuf1)
  sbufs = (sbuf0, sbuf1)
  sem_ts = (sem_t0, sem_t1)
  sem_ss = (sem_s0, sem_s1)
  wid = lax.axis_index("s") * NUM_CORES + lax.axis_index("c")
  base = wid * ATOMS_PER_W

  iota = lax.iota(jnp.int32, 16)
  lane_first = iota == 0
  lane_last = iota == 15
  key_rotl = (iota + 15) & 15
  key_rotr = (iota + 1) & 15

  def start_chunk(c):
    slot = c % 2
    cbase = base + c * CHUNK
    d1 = pltpu.async_copy(
        types_hbm.at[pl.ds(cbase, CHUNK)], tbufs[slot], sem_ts[slot])
    d2 = pltpu.async_copy(
        sys_hbm.at[pl.ds(cbase, CHUNK)], sbufs[slot], sem_ss[slot])
    return (d1, d2)

  def compute(slot):
    tb = tbufs[slot]
    sb = sbufs[slot]

    @plsc.parallel_loop(0, VECS, 1, unroll=UNROLL)
    def vec_body(k):
      s = sb[pl.ds(k * 16, 16)]
      _, s_next = plsc.sort_key_val(key_rotl, s)
      _, s_prev = plsc.sort_key_val(key_rotr, s)
      t = tb[pl.ds(k * 16, 16)]
      vw = plsc.load_gather(wbuf, [t])
      cs = plsc.cumsum(vw)
      run_end = (s != s_next) | lane_last
      run_start = (s != s_prev) | lane_first
      plsc.addupdate_scatter(acc, [s], cs, mask=run_end)
      plsc.addupdate_scatter(acc, [s], vw - cs, mask=run_start)

  pending = [start_chunk(0), start_chunk(1)]
  pltpu.sync_copy(w_hbm, wbuf)

  @plsc.parallel_loop(0, N_SYSTEMS // 16, 1, unroll=8)
  def zero_body(i):
    acc[pl.ds(i * 16, 16)] = jnp.zeros((16,), jnp.float32)

  for c in range(NCHUNKS):
    for d in pending[c]:
      d.wait()
    compute(c % 2)
    if c + 2 < NCHUNKS:
      pending.append(start_chunk(c + 2))

  pltpu.sync_copy(acc, part_hbm.at[wid])


def _merge_body(p_ref, o_ref):
  o_ref[...] = jnp.sum(p_ref[...], axis=0, keepdims=True)


@jax.jit
def kernel(types, system_ids, weights):
  w_pad = jnp.zeros((WPAD,), jnp.float32).at[:N_TYPES].set(weights)

  sc_fn = pl.kernel(
      _sc_body,
      out_type=jax.ShapeDtypeStruct((NW, N_SYSTEMS), jnp.float32),
      mesh=plsc.VectorSubcoreMesh(core_axis_name="c", subcore_axis_name="s"),
      compiler_params=pltpu.CompilerParams(needs_layout_passes=False),
      scratch_types=[
          pltpu.VMEM((WPAD,), jnp.float32),
          pltpu.VMEM((CHUNK,), jnp.int32),
          pltpu.VMEM((CHUNK,), jnp.int32),
          pltpu.VMEM((CHUNK,), jnp.int32),
          pltpu.VMEM((CHUNK,), jnp.int32),
          pltpu.VMEM((N_SYSTEMS,), jnp.float32),
          pltpu.SemaphoreType.DMA,
          pltpu.SemaphoreType.DMA,
          pltpu.SemaphoreType.DMA,
          pltpu.SemaphoreType.DMA,
      ],
  )
  partials = sc_fn(types, system_ids, w_pad)

  merged = pl.pallas_call(
      _merge_body,
      out_shape=jax.ShapeDtypeStruct((1, N_SYSTEMS), jnp.float32),
  )(partials)
  return merged.reshape(N_SYSTEMS, 1)


# speedup vs baseline: 581.7092x; 1.0183x over previous
import jax
import jax.numpy as jnp
from jax import lax
from jax.experimental import pallas as pl
from jax.experimental.pallas import tpu as pltpu
from jax.experimental.pallas import tpu_sc as plsc

N_ATOMS = 2097152
N_TYPES = 100
N_SYSTEMS = 8192

NUM_CORES = 2
NUM_SUBCORES = 16
NW = NUM_CORES * NUM_SUBCORES
ATOMS_PER_W = N_ATOMS // NW
CHUNK = 16384
NCHUNKS = ATOMS_PER_W // CHUNK
VECS = CHUNK // 16
WPAD = 128
UNROLL = 8


def _sc_body(types_hbm, sys_hbm, w_hbm, part_hbm, wbuf, tbuf0, tbuf1,
             sbuf0, sbuf1, acc, sem_t0, sem_t1, sem_s0, sem_s1):
  tbufs = (tbuf0, tbuf1)
  sbufs = (sbuf0, sbuf1)
  sem_ts = (sem_t0, sem_t1)
  sem_ss = (sem_s0, sem_s1)
  wid = lax.axis_index("s") * NUM_CORES + lax.axis_index("c")
  base = wid * ATOMS_PER_W

  def start_chunk(c):
    slot = c % 2
    cbase = base + c * CHUNK
    d1 = pltpu.async_copy(
        types_hbm.at[pl.ds(cbase, CHUNK)], tbufs[slot], sem_ts[slot])
    d2 = pltpu.async_copy(
        sys_hbm.at[pl.ds(cbase, CHUNK)], sbufs[slot], sem_ss[slot])
    return (d1, d2)

  def compute(slot):
    tb = tbufs[slot]
    sb = sbufs[slot]

    @plsc.parallel_loop(0, VECS, 1, unroll=UNROLL)
    def vec_body(k):
      s = sb[pl.ds(k * 16, 16)]
      cnt, run_end = plsc.scan_count(s)
      run_start = cnt == 1
      t = tb[pl.ds(k * 16, 16)]
      vw = plsc.load_gather(wbuf, [t])
      cs = plsc.cumsum(vw)
      plsc.addupdate_scatter(acc, [s], cs, mask=run_end)
      plsc.addupdate_scatter(acc, [s], vw - cs, mask=run_start)

  pending = [start_chunk(0), start_chunk(1)]
  pltpu.sync_copy(w_hbm, wbuf)

  @plsc.parallel_loop(0, N_SYSTEMS // 16, 1, unroll=8)
  def zero_body(i):
    acc[pl.ds(i * 16, 16)] = jnp.zeros((16,), jnp.float32)

  for c in range(NCHUNKS):
    for d in pending[c]:
      d.wait()
    compute(c % 2)
    if c + 2 < NCHUNKS:
      pending.append(start_chunk(c + 2))

  pltpu.sync_copy(acc, part_hbm.at[wid])


def _merge_body(p_ref, o_ref):
  o_ref[...] = jnp.sum(p_ref[...], axis=0, keepdims=True)


@jax.jit
def kernel(types, system_ids, weights):
  w_pad = jnp.zeros((WPAD,), jnp.float32).at[:N_TYPES].set(weights)

  sc_fn = pl.kernel(
      _sc_body,
      out_type=jax.ShapeDtypeStruct((NW, N_SYSTEMS), jnp.float32),
      mesh=plsc.VectorSubcoreMesh(core_axis_name="c", subcore_axis_name="s"),
      compiler_params=pltpu.CompilerParams(needs_layout_passes=False),
      scratch_types=[
          pltpu.VMEM((WPAD,), jnp.float32),
          pltpu.VMEM((CHUNK,), jnp.int32),
          pltpu.VMEM((CHUNK,), jnp.int32),
          pltpu.VMEM((CHUNK,), jnp.int32),
          pltpu.VMEM((CHUNK,), jnp.int32),
          pltpu.VMEM((N_SYSTEMS,), jnp.float32),
          pltpu.SemaphoreType.DMA,
          pltpu.SemaphoreType.DMA,
          pltpu.SemaphoreType.DMA,
          pltpu.SemaphoreType.DMA,
      ],
  )
  partials = sc_fn(types, system_ids, w_pad)

  merged = pl.pallas_call(
      _merge_body,
      out_shape=jax.ShapeDtypeStruct((1, N_SYSTEMS), jnp.float32),
  )(partials)
  return merged.reshape(N_SYSTEMS, 1)


# speedup vs baseline: 595.6207x; 1.0239x over previous
import jax
import jax.numpy as jnp
from jax import lax
from jax.experimental import pallas as pl
from jax.experimental.pallas import tpu as pltpu
from jax.experimental.pallas import tpu_sc as plsc

N_ATOMS = 2097152
N_TYPES = 100
N_SYSTEMS = 8192

NUM_CORES = 2
NUM_SUBCORES = 16
NW = NUM_CORES * NUM_SUBCORES
ATOMS_PER_W = N_ATOMS // NW
CHUNK = 16384
NCHUNKS = ATOMS_PER_W // CHUNK
VECS = CHUNK // 16
UNROLL = 8


def _sc_body(types_hbm, sys_hbm, w_hbm, part_hbm, wbuf, tbuf, sbuf, acc,
             sem_t, sem_s):
  wid = lax.axis_index("s") * NUM_CORES + lax.axis_index("c")
  base = wid * ATOMS_PER_W

  def chunk_refs(c):
    off = (c % 2) * CHUNK
    cbase = base + c * CHUNK
    slot = c % 2
    return (
        (types_hbm.at[pl.ds(cbase, CHUNK)], tbuf.at[pl.ds(off, CHUNK)],
         sem_t.at[slot]),
        (sys_hbm.at[pl.ds(cbase, CHUNK)], sbuf.at[pl.ds(off, CHUNK)],
         sem_s.at[slot]),
    )

  def start_chunk(c):
    for src, dst, sem in chunk_refs(c):
      pltpu.async_copy(src, dst, sem)

  start_chunk(0)
  start_chunk(1)
  pltpu.sync_copy(w_hbm, wbuf)

  @plsc.parallel_loop(0, N_SYSTEMS // 16, 1, unroll=8)
  def zero_body(i):
    acc[pl.ds(i * 16, 16)] = jnp.zeros((16,), jnp.float32)

  def chunk_body(c, carry):
    for src, dst, sem in chunk_refs(c):
      pltpu.make_async_copy(src, dst, sem).wait()
    off = (c % 2) * CHUNK

    @plsc.parallel_loop(0, VECS, 1, unroll=UNROLL)
    def vec_body(k):
      s = sbuf[pl.ds(off + k * 16, 16)]
      cnt, run_end = plsc.scan_count(s)
      run_start = cnt == 1
      t = tbuf[pl.ds(off + k * 16, 16)]
      vw = plsc.load_gather(wbuf, [t])
      cs = plsc.cumsum(vw)
      plsc.addupdate_scatter(acc, [s], cs, mask=run_end)
      plsc.addupdate_scatter(acc, [s], vw - cs, mask=run_start)

    @pl.when(c + 2 < NCHUNKS)
    def _():
      start_chunk(c + 2)

    return carry

  lax.fori_loop(0, NCHUNKS, chunk_body, 0)
  pltpu.sync_copy(acc, part_hbm.at[wid])


def _merge_body(p_ref, o_ref):
  o_ref[...] = jnp.sum(p_ref[...], axis=0, keepdims=True)


@jax.jit
def kernel(types, system_ids, weights):
  sc_fn = pl.kernel(
      _sc_body,
      out_type=jax.ShapeDtypeStruct((NW, N_SYSTEMS), jnp.float32),
      mesh=plsc.VectorSubcoreMesh(core_axis_name="c", subcore_axis_name="s"),
      compiler_params=pltpu.CompilerParams(needs_layout_passes=False),
      scratch_types=[
          pltpu.VMEM((N_TYPES,), jnp.float32),
          pltpu.VMEM((2 * CHUNK,), jnp.int32),
          pltpu.VMEM((2 * CHUNK,), jnp.int32),
          pltpu.VMEM((N_SYSTEMS,), jnp.float32),
          pltpu.SemaphoreType.DMA((2,)),
          pltpu.SemaphoreType.DMA((2,)),
      ],
  )
  partials = sc_fn(types, system_ids, weights)

  merged = pl.pallas_call(
      _merge_body,
      out_shape=jax.ShapeDtypeStruct((1, N_SYSTEMS), jnp.float32),
  )(partials)
  return merged.reshape(N_SYSTEMS, 1)


# speedup vs baseline: 612.8498x; 1.0289x over previous
import jax
import jax.numpy as jnp
from jax import lax
from jax.experimental import pallas as pl
from jax.experimental.pallas import tpu as pltpu
from jax.experimental.pallas import tpu_sc as plsc

N_ATOMS = 2097152
N_TYPES = 100
N_SYSTEMS = 8192

NUM_CORES = 2
NUM_SUBCORES = 16
NW = NUM_CORES * NUM_SUBCORES
ATOMS_PER_W = N_ATOMS // NW
CHUNK = 8192
NCHUNKS = ATOMS_PER_W // CHUNK
VECS = CHUNK // 16
UNROLL = 8


def _sc_body(types_hbm, sys_hbm, w_hbm, part_hbm, wbuf, tbuf, sbuf, acc,
             sem_t, sem_s):
  wid = lax.axis_index("s") * NUM_CORES + lax.axis_index("c")
  base = wid * ATOMS_PER_W

  def chunk_refs(c):
    off = (c % 2) * CHUNK
    cbase = base + c * CHUNK
    slot = c % 2
    return (
        (types_hbm.at[pl.ds(cbase, CHUNK)], tbuf.at[pl.ds(off, CHUNK)],
         sem_t.at[slot]),
        (sys_hbm.at[pl.ds(cbase, CHUNK)], sbuf.at[pl.ds(off, CHUNK)],
         sem_s.at[slot]),
    )

  def start_chunk(c):
    for src, dst, sem in chunk_refs(c):
      pltpu.async_copy(src, dst, sem)

  start_chunk(0)
  start_chunk(1)
  pltpu.sync_copy(w_hbm, wbuf)

  @plsc.parallel_loop(0, N_SYSTEMS // 16, 1, unroll=8)
  def zero_body(i):
    acc[pl.ds(i * 16, 16)] = jnp.zeros((16,), jnp.float32)

  def chunk_body(c, carry):
    for src, dst, sem in chunk_refs(c):
      pltpu.make_async_copy(src, dst, sem).wait()
    off = (c % 2) * CHUNK

    @plsc.parallel_loop(0, VECS, 1, unroll=UNROLL)
    def vec_body(k):
      s = sbuf[pl.ds(off + k * 16, 16)]
      cnt, run_end = plsc.scan_count(s)
      run_start = cnt == 1
      t = tbuf[pl.ds(off + k * 16, 16)]
      vw = plsc.load_gather(wbuf, [t])
      cs = plsc.cumsum(vw)
      plsc.addupdate_scatter(acc, [s], cs, mask=run_end)
      plsc.addupdate_scatter(acc, [s], vw - cs, mask=run_start)

    @pl.when(c + 2 < NCHUNKS)
    def _():
      start_chunk(c + 2)

    return carry

  lax.fori_loop(0, NCHUNKS, chunk_body, 0)
  pltpu.sync_copy(acc, part_hbm.at[wid])


def _merge_body(p_ref, o_ref):
  o_ref[...] = jnp.sum(p_ref[...], axis=0, keepdims=True)


@jax.jit
def kernel(types, system_ids, weights):
  sc_fn = pl.kernel(
      _sc_body,
      out_type=jax.ShapeDtypeStruct((NW, N_SYSTEMS), jnp.float32),
      mesh=plsc.VectorSubcoreMesh(core_axis_name="c", subcore_axis_name="s"),
      compiler_params=pltpu.CompilerParams(needs_layout_passes=False),
      scratch_types=[
          pltpu.VMEM((N_TYPES,), jnp.float32),
          pltpu.VMEM((2 * CHUNK,), jnp.int32),
          pltpu.VMEM((2 * CHUNK,), jnp.int32),
          pltpu.VMEM((N_SYSTEMS,), jnp.float32),
          pltpu.SemaphoreType.DMA((2,)),
          pltpu.SemaphoreType.DMA((2,)),
      ],
  )
  partials = sc_fn(types, system_ids, weights)

  merged = pl.pallas_call(
      _merge_body,
      out_shape=jax.ShapeDtypeStruct((1, N_SYSTEMS), jnp.float32),
  )(partials)
  return merged.reshape(N_SYSTEMS, 1)
